# Initial kernel scaffold; baseline (speedup 1.0000x reference)
import jax
import jax.numpy as jnp
from jax.experimental import pallas as pl


def kernel(node_subgraph, edge_index, edge_weight, feat_full, label_full, W0a, b0a, s0a, o0a, W0b, b0b, s0b, o0b, W1a, b1a, s1a, o1a, W1b, b1b, s1b, o1b, Wc, bc):
    raise NotImplementedError("write your pallas kernel here")


# speedup vs baseline: 1.5515x; 1.5515x over previous
import functools

import jax
import jax.numpy as jnp
from jax import lax
from jax.experimental import pallas as pl
from jax.experimental.pallas import tpu as pltpu
from jax.experimental.pallas import tpu_sc as plsc

N_SUB = 10000
E = 160000
N_TILES = 16
N_WORKERS = 32

ROWS_PER_W = 384
NSUB_PAD = N_WORKERS * ROWS_PER_W

_sc_mesh = plsc.VectorSubcoreMesh(core_axis_name="c", subcore_axis_name="s")


@functools.partial(
    pl.kernel,
    out_type=(
        jax.ShapeDtypeStruct((NSUB_PAD, 256), jnp.float32),
        jax.ShapeDtypeStruct((NSUB_PAD, 128), jnp.float32),
    ),
    mesh=_sc_mesh,
    scratch_types=[
        pltpu.VMEM((3, 128), jnp.int32),
        pltpu.VMEM((128, 256), jnp.float32),
        pltpu.VMEM((128, 128), jnp.float32),
        pltpu.SemaphoreType.DMA,
        pltpu.SemaphoreType.DMA,
    ],
)
def _sc_gather(idx_hbm, feat_hbm, label_hbm, outf_hbm, outl_hbm,
               idx_v, rf_v, rl_v, semf, seml):
    cid = lax.axis_index("c")
    sid = lax.axis_index("s")
    wid = sid * 2 + cid
    pltpu.sync_copy(idx_hbm.at[wid], idx_v)
    base = wid * ROWS_PER_W
    for j in range(3):
        pltpu.async_copy(feat_hbm.at[idx_v.at[j]], rf_v, semf).wait()
        pltpu.sync_copy(rf_v, outf_hbm.at[pl.ds(base + j * 128, 128)])
        pltpu.async_copy(label_hbm.at[idx_v.at[j]], rl_v, seml).wait()
        pltpu.sync_copy(rl_v, outl_hbm.at[pl.ds(base + j * 128, 128)])


EPT = 10240
NB = EPT // 128
N_ACC = 10240


def _make_spmm(C, mul, off_fn):
    C_sc = C // 2

    @functools.partial(
        pl.kernel,
        out_type=jax.ShapeDtypeStruct((C, N_ACC, 64), jnp.float32),
        mesh=_sc_mesh,
        compiler_params=pltpu.CompilerParams(use_tc_tiling_on_sc=False),
        scratch_types=[
            pltpu.VMEM((NB, 128), jnp.int32),
            pltpu.VMEM((NB, 128), jnp.int32),
            pltpu.VMEM((128, 16), jnp.float32),
            pltpu.VMEM((NB, 128), jnp.int32),
            pltpu.VMEM((2, 128, 64), jnp.float32),
            pltpu.VMEM_SHARED((N_ACC, 64), jnp.float32),
            pltpu.SemaphoreType.DMA,
            pltpu.SemaphoreType.DMA,
        ],
    )
    def spmm(feat_hbm, src_hbm, dst_hbm, ew_hbm, out_hbm,
             src_v, dst_v, ew_v, gidx_v, rows_v, acc_sh, gsem, esem):
        cid = lax.axis_index("c")
        sid = lax.axis_index("s")
        pltpu.sync_copy(src_hbm.at[sid], src_v)
        pltpu.sync_copy(dst_hbm.at[sid], dst_v)

        for j in range(C_sc):
            c = 2 * j + cid
            off = off_fn(j, cid)
            def idx_body(i, carry):
                for k in range(8):
                    s = src_v[i, pl.ds(k * 16, 16)]
                    gidx_v[i, pl.ds(k * 16, 16)] = s * mul + off
                return carry
            lax.fori_loop(0, NB, idx_body, 0)

            def z_body(i, carry):
                for k in range(4):
                    rows_v[0, i, pl.ds(k * 16, 16)] = jnp.zeros(
                        (16,), jnp.float32)
                return carry
            lax.fori_loop(0, 128, z_body, 0)
            zbase = sid * 640
            for r in range(5):
                pltpu.sync_copy(rows_v.at[0],
                                acc_sh.at[pl.ds(zbase + r * 128, 128)])
            plsc.subcore_barrier()

            def b_body(b, carry):
                cp_ew = pltpu.async_copy(ew_hbm.at[sid].at[b], ew_v, esem)
                cp_rows = pltpu.async_copy(feat_hbm.at[gidx_v.at[b]],
                                           rows_v.at[0], gsem)
                cp_ew.wait()
                cp_rows.wait()

                def e_body(e, carry2):
                    ewb = ew_v[e, :]
                    for k in range(4):
                        rows_v[0, e, pl.ds(k * 16, 16)] = (
                            rows_v[0, e, pl.ds(k * 16, 16)] * ewb)
                    return carry2
                lax.fori_loop(0, 128, e_body, 0)
                pltpu.sync_copy(rows_v.at[0], acc_sh.at[dst_v.at[b]],
                                add=True)
                return carry
            lax.fori_loop(0, NB, b_body, 0)
            plsc.subcore_barrier()
            pltpu.sync_copy(acc_sh.at[pl.ds(zbase, 640)],
                            out_hbm.at[c].at[pl.ds(zbase, 640)])

    return spmm


_spmm_l0 = _make_spmm(4, 4, lambda j, cid: 2 * j + cid)
_spmm_l1 = _make_spmm(8, 2, lambda j, cid: j * 20000 + cid)


RB = 1000
_PREC = jax.lax.Precision.HIGHEST


def _ft(h_lin, b, s, o):
    h = jax.nn.relu(h_lin + b)
    mean = jnp.mean(h, axis=1, keepdims=True)
    d = h - mean
    var = jnp.mean(d * d, axis=1, keepdims=True) + 1e-9
    return d * s * lax.rsqrt(var) + o


def _dot(a, b):
    return jnp.dot(a, b, preferred_element_type=jnp.float32,
                   precision=_PREC)


def _tc1_body(feat_ref, agg_ref, w0a_ref, b0a_ref, s0a_ref, o0a_ref,
              w0b_ref, b0b_ref, s0b_ref, o0b_ref,
              w1a_ref, b1a_ref, s1a_ref, o1a_ref, w1b_ref,
              p01_ref, y_ref):
    x = feat_ref[...]
    p0 = _ft(_dot(x, w0a_ref[...]), b0a_ref[...], s0a_ref[...],
             o0a_ref[...])
    h1 = _dot(agg_ref[0], w0b_ref[0:128]) + _dot(agg_ref[1],
                                                 w0b_ref[128:256])
    p1 = _ft(h1, b0b_ref[...], s0b_ref[...], o0b_ref[...])
    p01 = _ft(_dot(p0, w1a_ref[0:512]) + _dot(p1, w1a_ref[512:1024]),
              b1a_ref[...], s1a_ref[...], o1a_ref[...])
    p01_ref[...] = p01
    y = _dot(p0, w1b_ref[0:512]) + _dot(p1, w1b_ref[512:1024])
    for c in range(4):
        y_ref[c] = y[:, c * 128:(c + 1) * 128]


def _tc2_body(p01_ref, z_ref, b1b_ref, s1b_ref, o1b_ref, wc_ref, bc_ref,
              out_ref):
    z = jnp.concatenate([z_ref[0], z_ref[1], z_ref[2], z_ref[3]], axis=1)
    p11 = _ft(z, b1b_ref[...], s1b_ref[...], o1b_ref[...])
    p01 = p01_ref[...]
    nrm2 = (jnp.sum(p01 * p01, axis=1, keepdims=True)
            + jnp.sum(p11 * p11, axis=1, keepdims=True))
    nrm = jnp.maximum(jnp.sqrt(nrm2), 1e-12)
    pred = (_dot(p01, wc_ref[0:512]) + _dot(p11, wc_ref[512:1024])) / nrm
    out_ref[...] = pred + bc_ref[...]


def _full(shape):
    return pl.BlockSpec(shape, lambda i: (0,) * len(shape))


def _tc1(feat_pad, agg0, w0a, b0a, s0a, o0a, w0b, b0b, s0b, o0b,
         w1a, b1a, s1a, o1a, w1b):
    grid = (N_SUB // RB,)
    return pl.pallas_call(
        _tc1_body,
        grid=grid,
        in_specs=[
            pl.BlockSpec((RB, 256), lambda i: (i, 0)),
            pl.BlockSpec((2, RB, 128), lambda i: (0, i, 0)),
            _full((256, 512)), _full((1, 512)), _full((1, 512)),
            _full((1, 512)),
            _full((256, 512)), _full((1, 512)), _full((1, 512)),
            _full((1, 512)),
            _full((1024, 512)), _full((1, 512)), _full((1, 512)),
            _full((1, 512)), _full((1024, 512)),
        ],
        out_specs=[
            pl.BlockSpec((RB, 512), lambda i: (i, 0)),
            pl.BlockSpec((4, RB, 128), lambda i: (0, i, 0)),
        ],
        out_shape=[
            jax.ShapeDtypeStruct((N_SUB, 512), jnp.float32),
            jax.ShapeDtypeStruct((4, N_SUB, 128), jnp.float32),
        ],
    )(feat_pad, agg0, w0a, b0a, s0a, o0a, w0b, b0b, s0b, o0b,
      w1a, b1a, s1a, o1a, w1b)


def _tc2(p01, z, b1b, s1b, o1b, wc, bc):
    grid = (N_SUB // RB,)
    return pl.pallas_call(
        _tc2_body,
        grid=grid,
        in_specs=[
            pl.BlockSpec((RB, 512), lambda i: (i, 0)),
            pl.BlockSpec((4, RB, 128), lambda i: (0, i, 0)),
            _full((1, 512)), _full((1, 512)), _full((1, 512)),
            _full((1024, 128)), _full((1, 128)),
        ],
        out_specs=pl.BlockSpec((RB, 128), lambda i: (i, 0)),
        out_shape=jax.ShapeDtypeStruct((N_SUB, 128), jnp.float32),
    )(p01, z, b1b, s1b, o1b, wc, bc)


def kernel(node_subgraph, edge_index, edge_weight, feat_full, label_full,
           W0a, b0a, s0a, o0a, W0b, b0b, s0b, o0b,
           W1a, b1a, s1a, o1a, W1b, b1b, s1b, o1b, Wc, bc):
    idx_pad = jnp.zeros((NSUB_PAD,), jnp.int32).at[:N_SUB].set(
        node_subgraph.astype(jnp.int32)).reshape(N_WORKERS, 3, 128)
    src = edge_index[0].astype(jnp.int32)
    dst = edge_index[1].astype(jnp.int32)
    epad = N_TILES * EPT
    src3 = jnp.zeros((epad,), jnp.int32).at[:E].set(src).reshape(
        N_TILES, NB, 128)
    dst3 = jnp.zeros((epad,), jnp.int32).at[:E].set(dst).reshape(
        N_TILES, NB, 128)
    ew_pad = jnp.zeros((epad,), jnp.float32).at[:E].set(edge_weight)
    ew3 = jnp.broadcast_to(ew_pad[:, None], (epad, 16)).reshape(
        N_TILES, NB, 128, 16)

    def row(v):
        return v.reshape(1, -1)

    feat_pad, label_pad = _sc_gather(idx_pad, feat_full, label_full)
    label_subg = label_pad[:N_SUB]

    agg0_64 = _spmm_l0(feat_pad.reshape(-1, 64), src3, dst3, ew3)
    agg0 = agg0_64.reshape(2, 2, N_ACC, 64).transpose(0, 2, 1, 3).reshape(
        2, N_ACC, 128)

    p01, y = _tc1(feat_pad, agg0, W0a, row(b0a), row(s0a), row(o0a),
                  W0b, row(b0b), row(s0b), row(o0b),
                  W1a, row(b1a), row(s1a), row(o1a), W1b)

    z64 = _spmm_l1(y.reshape(-1, 64), src3, dst3, ew3)
    z = z64.reshape(4, 2, N_ACC, 64).transpose(0, 2, 1, 3).reshape(
        4, N_ACC, 128)

    pred = _tc2(p01, z, row(b1b), row(s1b), row(o1b), Wc, row(bc))
    return (pred, label_subg, label_subg)
